# baseline (speedup 1.0000x reference)
import functools

import jax
import jax.numpy as jnp
from jax import lax
from jax.experimental import pallas as pl
from jax.experimental.pallas import tpu as pltpu
from jax.experimental.pallas import tpu_sc as plsc

_B = 1024
_H = 64
_NSC = 2
_HALF = _B // _NSC

_BN = 512
_NBUF = 16

_scalar_mesh = plsc.ScalarSubcoreMesh(axis_name="core", num_cores=_NSC)


@functools.partial(
    pl.kernel,
    mesh=_scalar_mesh,
    out_type=jax.ShapeDtypeStruct((_B, _H), jnp.float32),
    scratch_types=[
        pltpu.SMEM((_HALF,), jnp.int32),
        pltpu.SemaphoreType.DMA,
        pltpu.SemaphoreType.DMA,
    ],
)
def _sc_gather(table_hbm, idx_hbm, out_hbm, idx_s, isem, gsem):
    cid = lax.axis_index("core")
    base = cid * _HALF
    pltpu.async_copy(idx_hbm.at[pl.ds(base, _HALF)], idx_s, isem).wait()

    @pl.loop(0, _HALF)
    def _(i):
        pltpu.make_async_copy(
            table_hbm.at[idx_s[i]], out_hbm.at[base + i], gsem
        ).start()

    @pl.loop(0, _HALF)
    def _(i):
        pltpu.make_async_copy(
            table_hbm.at[0], out_hbm.at[base], gsem
        ).wait()


def _make_mm_body(ng, v_tail):
    def _mm_body(h_ref, wt_ref, o_hbm, ht_ref, obuf, sems):
        i = pl.program_id(0)
        slot = lax.rem(i, _NBUF)

        @pl.when(i == 0)
        def _():
            ht_ref[...] = jnp.transpose(h_ref[...]).astype(jnp.bfloat16)

        @pl.when(i >= _NBUF)
        def _():
            pltpu.make_async_copy(
                obuf.at[slot],
                o_hbm.at[pl.ds((i - _NBUF) * _BN, _BN)],
                sems.at[slot],
            ).wait()

        obuf[slot] = lax.dot_general(
            wt_ref[...].astype(jnp.bfloat16),
            ht_ref[...],
            dimension_numbers=(((0,), (0,)), ((), ())),
            preferred_element_type=jnp.float32,
        )

        @pl.when(i < ng - 1)
        def _():
            pltpu.make_async_copy(
                obuf.at[slot],
                o_hbm.at[pl.ds(i * _BN, _BN)],
                sems.at[slot],
            ).start()

        @pl.when(i == ng - 1)
        def _():
            pltpu.make_async_copy(
                obuf.at[slot, pl.ds(0, v_tail)],
                o_hbm.at[pl.ds(i * _BN, v_tail)],
                sems.at[slot],
            ).start()
            for k in range(_NBUF - 1):
                j = ng - _NBUF + k
                pltpu.make_async_copy(
                    obuf.at[j % _NBUF],
                    o_hbm.at[pl.ds(j * _BN, _BN)],
                    sems.at[j % _NBUF],
                ).wait()
            pltpu.make_async_copy(
                obuf.at[slot, pl.ds(0, v_tail)],
                o_hbm.at[pl.ds(i * _BN, v_tail)],
                sems.at[slot],
            ).wait()

    return _mm_body


def kernel(x, emb, W):
    xi = x.astype(jnp.int32)
    h = _sc_gather(emb, xi)
    V = W.shape[0]
    wt = W.T
    ng = pl.cdiv(V, _BN)
    v_tail = V - (ng - 1) * _BN

    lt = pl.pallas_call(
        _make_mm_body(ng, v_tail),
        grid=(ng,),
        in_specs=[
            pl.BlockSpec((_B, _H), lambda i: (0, 0)),
            pl.BlockSpec((_H, _BN), lambda i: (0, i)),
        ],
        out_specs=pl.BlockSpec(memory_space=pl.ANY),
        out_shape=jax.ShapeDtypeStruct((V, _B), jnp.float32),
        scratch_shapes=[
            pltpu.VMEM((_H, _B), jnp.bfloat16),
            pltpu.VMEM((_NBUF, _BN, _B), jnp.float32),
            pltpu.SemaphoreType.DMA((_NBUF,)),
        ],
        compiler_params=pltpu.CompilerParams(
            dimension_semantics=("arbitrary",),
        ),
    )(h, wt)
    return lt.T


# speedup vs baseline: 1.2385x; 1.2385x over previous
import functools

import jax
import jax.numpy as jnp
from jax import lax
from jax.experimental import pallas as pl
from jax.experimental.pallas import tpu as pltpu
from jax.experimental.pallas import tpu_sc as plsc

_B = 1024
_H = 64
_NSC = 2
_HALF = _B // _NSC

_BN = 1024
_NBUF = 8

_scalar_mesh = plsc.ScalarSubcoreMesh(axis_name="core", num_cores=_NSC)


@functools.partial(
    pl.kernel,
    mesh=_scalar_mesh,
    out_type=jax.ShapeDtypeStruct((_B, _H), jnp.float32),
    scratch_types=[
        pltpu.SMEM((_HALF,), jnp.int32),
        pltpu.SemaphoreType.DMA,
        pltpu.SemaphoreType.DMA,
    ],
)
def _sc_gather(table_hbm, idx_hbm, out_hbm, idx_s, isem, gsem):
    cid = lax.axis_index("core")
    base = cid * _HALF
    pltpu.async_copy(idx_hbm.at[pl.ds(base, _HALF)], idx_s, isem).wait()

    @pl.loop(0, _HALF)
    def _(i):
        pltpu.make_async_copy(
            table_hbm.at[idx_s[i]], out_hbm.at[base + i], gsem
        ).start()

    @pl.loop(0, _HALF)
    def _(i):
        pltpu.make_async_copy(
            table_hbm.at[0], out_hbm.at[base], gsem
        ).wait()


def _make_mm_body(ng, v_tail):
    def _mm_body(h_ref, wt_ref, o_hbm, ht_ref, obuf, sems):
        i = pl.program_id(0)
        slot = lax.rem(i, _NBUF)

        @pl.when(i == 0)
        def _():
            ht_ref[...] = jnp.transpose(h_ref[...]).astype(jnp.bfloat16)

        @pl.when(i >= _NBUF)
        def _():
            pltpu.make_async_copy(
                obuf.at[slot],
                o_hbm.at[pl.ds((i - _NBUF) * _BN, _BN)],
                sems.at[slot],
            ).wait()

        obuf[slot] = lax.dot_general(
            wt_ref[...].astype(jnp.bfloat16),
            ht_ref[...],
            dimension_numbers=(((0,), (0,)), ((), ())),
            preferred_element_type=jnp.float32,
        )

        @pl.when(i < ng - 1)
        def _():
            pltpu.make_async_copy(
                obuf.at[slot],
                o_hbm.at[pl.ds(i * _BN, _BN)],
                sems.at[slot],
            ).start()

        @pl.when(i == ng - 1)
        def _():
            pltpu.make_async_copy(
                obuf.at[slot, pl.ds(0, v_tail)],
                o_hbm.at[pl.ds(i * _BN, v_tail)],
                sems.at[slot],
            ).start()
            for k in range(_NBUF - 1):
                j = ng - _NBUF + k
                pltpu.make_async_copy(
                    obuf.at[j % _NBUF],
                    o_hbm.at[pl.ds(j * _BN, _BN)],
                    sems.at[j % _NBUF],
                ).wait()
            pltpu.make_async_copy(
                obuf.at[slot, pl.ds(0, v_tail)],
                o_hbm.at[pl.ds(i * _BN, v_tail)],
                sems.at[slot],
            ).wait()

    return _mm_body


def kernel(x, emb, W):
    xi = x.astype(jnp.int32)
    h = _sc_gather(emb, xi)
    V = W.shape[0]
    wt = W.T
    ng = pl.cdiv(V, _BN)
    v_tail = V - (ng - 1) * _BN

    lt = pl.pallas_call(
        _make_mm_body(ng, v_tail),
        grid=(ng,),
        in_specs=[
            pl.BlockSpec((_B, _H), lambda i: (0, 0)),
            pl.BlockSpec((_H, _BN), lambda i: (0, i)),
        ],
        out_specs=pl.BlockSpec(memory_space=pl.ANY),
        out_shape=jax.ShapeDtypeStruct((V, _B), jnp.float32),
        scratch_shapes=[
            pltpu.VMEM((_H, _B), jnp.bfloat16),
            pltpu.VMEM((_NBUF, _BN, _B), jnp.float32),
            pltpu.SemaphoreType.DMA((_NBUF,)),
        ],
        compiler_params=pltpu.CompilerParams(
            dimension_semantics=("arbitrary",),
        ),
    )(h, wt)
    return lt.T


# speedup vs baseline: 1.2595x; 1.0170x over previous
import functools

import jax
import jax.numpy as jnp
from jax import lax
from jax.experimental import pallas as pl
from jax.experimental.pallas import tpu as pltpu
from jax.experimental.pallas import tpu_sc as plsc

_B = 1024
_H = 64
_NSC = 2
_HALF = _B // _NSC

_BN = 2048
_NBUF = 6

_scalar_mesh = plsc.ScalarSubcoreMesh(axis_name="core", num_cores=_NSC)


@functools.partial(
    pl.kernel,
    mesh=_scalar_mesh,
    out_type=jax.ShapeDtypeStruct((_B, _H), jnp.float32),
    scratch_types=[
        pltpu.SMEM((_HALF,), jnp.int32),
        pltpu.SemaphoreType.DMA,
        pltpu.SemaphoreType.DMA,
    ],
)
def _sc_gather(table_hbm, idx_hbm, out_hbm, idx_s, isem, gsem):
    cid = lax.axis_index("core")
    base = cid * _HALF
    pltpu.async_copy(idx_hbm.at[pl.ds(base, _HALF)], idx_s, isem).wait()

    @pl.loop(0, _HALF)
    def _(i):
        pltpu.make_async_copy(
            table_hbm.at[idx_s[i]], out_hbm.at[base + i], gsem
        ).start()

    @pl.loop(0, _HALF)
    def _(i):
        pltpu.make_async_copy(
            table_hbm.at[0], out_hbm.at[base], gsem
        ).wait()


def _make_mm_body(ng, v_tail):
    def _mm_body(h_ref, wt_ref, o_hbm, ht_ref, obuf, sems):
        i = pl.program_id(0)
        slot = lax.rem(i, _NBUF)

        @pl.when(i == 0)
        def _():
            ht_ref[...] = jnp.transpose(h_ref[...]).astype(jnp.bfloat16)

        @pl.when(i >= _NBUF)
        def _():
            pltpu.make_async_copy(
                obuf.at[slot],
                o_hbm.at[pl.ds((i - _NBUF) * _BN, _BN)],
                sems.at[slot],
            ).wait()

        obuf[slot] = lax.dot_general(
            wt_ref[...].astype(jnp.bfloat16),
            ht_ref[...],
            dimension_numbers=(((0,), (0,)), ((), ())),
            preferred_element_type=jnp.float32,
        )

        @pl.when(i < ng - 1)
        def _():
            pltpu.make_async_copy(
                obuf.at[slot],
                o_hbm.at[pl.ds(i * _BN, _BN)],
                sems.at[slot],
            ).start()

        @pl.when(i == ng - 1)
        def _():
            pltpu.make_async_copy(
                obuf.at[slot, pl.ds(0, v_tail)],
                o_hbm.at[pl.ds(i * _BN, v_tail)],
                sems.at[slot],
            ).start()
            for k in range(_NBUF - 1):
                j = ng - _NBUF + k
                pltpu.make_async_copy(
                    obuf.at[j % _NBUF],
                    o_hbm.at[pl.ds(j * _BN, _BN)],
                    sems.at[j % _NBUF],
                ).wait()
            pltpu.make_async_copy(
                obuf.at[slot, pl.ds(0, v_tail)],
                o_hbm.at[pl.ds(i * _BN, v_tail)],
                sems.at[slot],
            ).wait()

    return _mm_body


def kernel(x, emb, W):
    xi = x.astype(jnp.int32)
    h = _sc_gather(emb, xi)
    V = W.shape[0]
    wt = W.T
    ng = pl.cdiv(V, _BN)
    v_tail = V - (ng - 1) * _BN

    lt = pl.pallas_call(
        _make_mm_body(ng, v_tail),
        grid=(ng,),
        in_specs=[
            pl.BlockSpec((_B, _H), lambda i: (0, 0)),
            pl.BlockSpec((_H, _BN), lambda i: (0, i)),
        ],
        out_specs=pl.BlockSpec(memory_space=pl.ANY),
        out_shape=jax.ShapeDtypeStruct((V, _B), jnp.float32),
        scratch_shapes=[
            pltpu.VMEM((_H, _B), jnp.bfloat16),
            pltpu.VMEM((_NBUF, _BN, _B), jnp.float32),
            pltpu.SemaphoreType.DMA((_NBUF,)),
        ],
        compiler_params=pltpu.CompilerParams(
            dimension_semantics=("arbitrary",),
        ),
    )(h, wt)
    return lt.T


# speedup vs baseline: 1.2694x; 1.0079x over previous
import functools

import jax
import jax.numpy as jnp
from jax import lax
from jax.experimental import pallas as pl
from jax.experimental.pallas import tpu as pltpu
from jax.experimental.pallas import tpu_sc as plsc

_B = 1024
_H = 64
_NSC = 2
_HALF = _B // _NSC

_BN = 4096
_NBUF = 3

_scalar_mesh = plsc.ScalarSubcoreMesh(axis_name="core", num_cores=_NSC)


@functools.partial(
    pl.kernel,
    mesh=_scalar_mesh,
    out_type=jax.ShapeDtypeStruct((_B, _H), jnp.float32),
    scratch_types=[
        pltpu.SMEM((_HALF,), jnp.int32),
        pltpu.SemaphoreType.DMA,
        pltpu.SemaphoreType.DMA,
    ],
)
def _sc_gather(table_hbm, idx_hbm, out_hbm, idx_s, isem, gsem):
    cid = lax.axis_index("core")
    base = cid * _HALF
    pltpu.async_copy(idx_hbm.at[pl.ds(base, _HALF)], idx_s, isem).wait()

    @pl.loop(0, _HALF)
    def _(i):
        pltpu.make_async_copy(
            table_hbm.at[idx_s[i]], out_hbm.at[base + i], gsem
        ).start()

    @pl.loop(0, _HALF)
    def _(i):
        pltpu.make_async_copy(
            table_hbm.at[0], out_hbm.at[base], gsem
        ).wait()


def _make_mm_body(ng, v_tail):
    def _mm_body(h_ref, wt_ref, o_hbm, ht_ref, obuf, sems):
        i = pl.program_id(0)
        slot = lax.rem(i, _NBUF)

        @pl.when(i == 0)
        def _():
            ht_ref[...] = jnp.transpose(h_ref[...]).astype(jnp.bfloat16)

        @pl.when(i >= _NBUF)
        def _():
            pltpu.make_async_copy(
                obuf.at[slot],
                o_hbm.at[pl.ds((i - _NBUF) * _BN, _BN)],
                sems.at[slot],
            ).wait()

        obuf[slot] = lax.dot_general(
            wt_ref[...].astype(jnp.bfloat16),
            ht_ref[...],
            dimension_numbers=(((0,), (0,)), ((), ())),
            preferred_element_type=jnp.float32,
        )

        @pl.when(i < ng - 1)
        def _():
            pltpu.make_async_copy(
                obuf.at[slot],
                o_hbm.at[pl.ds(i * _BN, _BN)],
                sems.at[slot],
            ).start()

        @pl.when(i == ng - 1)
        def _():
            pltpu.make_async_copy(
                obuf.at[slot, pl.ds(0, v_tail)],
                o_hbm.at[pl.ds(i * _BN, v_tail)],
                sems.at[slot],
            ).start()
            for k in range(_NBUF - 1):
                j = ng - _NBUF + k
                pltpu.make_async_copy(
                    obuf.at[j % _NBUF],
                    o_hbm.at[pl.ds(j * _BN, _BN)],
                    sems.at[j % _NBUF],
                ).wait()
            pltpu.make_async_copy(
                obuf.at[slot, pl.ds(0, v_tail)],
                o_hbm.at[pl.ds(i * _BN, v_tail)],
                sems.at[slot],
            ).wait()

    return _mm_body


def kernel(x, emb, W):
    xi = x.astype(jnp.int32)
    h = _sc_gather(emb, xi)
    V = W.shape[0]
    wt = W.T
    ng = pl.cdiv(V, _BN)
    v_tail = V - (ng - 1) * _BN

    lt = pl.pallas_call(
        _make_mm_body(ng, v_tail),
        grid=(ng,),
        in_specs=[
            pl.BlockSpec((_B, _H), lambda i: (0, 0)),
            pl.BlockSpec((_H, _BN), lambda i: (0, i)),
        ],
        out_specs=pl.BlockSpec(memory_space=pl.ANY),
        out_shape=jax.ShapeDtypeStruct((V, _B), jnp.float32),
        scratch_shapes=[
            pltpu.VMEM((_H, _B), jnp.bfloat16),
            pltpu.VMEM((_NBUF, _BN, _B), jnp.float32),
            pltpu.SemaphoreType.DMA((_NBUF,)),
        ],
        compiler_params=pltpu.CompilerParams(
            dimension_semantics=("arbitrary",),
        ),
    )(h, wt)
    return lt.T
